# Initial kernel scaffold; baseline (speedup 1.0000x reference)
import jax
import jax.numpy as jnp
from jax.experimental import pallas as pl


def kernel(x, edge_index, Wl1, bl1, Wr1, Wl2, bl2, Wr2, Wc, bc):
    raise NotImplementedError("write your pallas kernel here")


# speedup vs baseline: 4.7193x; 4.7193x over previous
import functools

import jax
import jax.numpy as jnp
from jax import lax
from jax.experimental import pallas as pl
from jax.experimental.pallas import tpu as pltpu
from jax.experimental.pallas import tpu_sc as plsc

N = 10000
E = 320000
D = 128
H = 64

_NC = 2
_NS = 16
_CPB = 128
_NCHP = 2560
_EPAD = _NCHP * _CPB
_WCH = _NCHP // (_NC * _NS)
_NR = 10112
_RPS = _NR // _NS


def _make_agg(W):
    mesh = plsc.VectorSubcoreMesh(core_axis_name="c", subcore_axis_name="s")

    @functools.partial(
        pl.kernel,
        mesh=mesh,
        compiler_params=pltpu.CompilerParams(use_tc_tiling_on_sc=False),
        out_type=jax.ShapeDtypeStruct((_NC, _NR, W), jnp.float32),
        scratch_types=[
            pltpu.VMEM((_WCH, _CPB), jnp.int32),
            pltpu.VMEM((_WCH, _CPB), jnp.int32),
            pltpu.VMEM((_CPB, W), jnp.float32),
            pltpu.VMEM_SHARED((_NR, W), jnp.float32),
        ],
    )
    def agg(y_hbm, src_hbm, dst_hbm, p_hbm, srcv, dstv, rows, acc):
        c = lax.axis_index("c")
        s = lax.axis_index("s")
        wid = s * _NC + c

        zeros16 = jnp.zeros((16,), jnp.float32)

        def zbody(r, carry):
            for k2 in range(W // 16):
                rows[r, pl.ds(k2 * 16, 16)] = zeros16
            return carry

        lax.fori_loop(0, _CPB, zbody, 0)
        full = _RPS // _CPB
        rem = _RPS - full * _CPB
        for t in range(full):
            pltpu.sync_copy(rows.at[pl.ds(0, _CPB)],
                            acc.at[pl.ds(s * _RPS + t * _CPB, _CPB)])
        if rem:
            pltpu.sync_copy(rows.at[pl.ds(0, rem)],
                            acc.at[pl.ds(s * _RPS + full * _CPB, rem)])

        pltpu.sync_copy(src_hbm.at[pl.ds(wid * _WCH, _WCH)], srcv)
        pltpu.sync_copy(dst_hbm.at[pl.ds(wid * _WCH, _WCH)], dstv)

        plsc.subcore_barrier()

        def body(j, carry):
            pltpu.sync_copy(y_hbm.at[srcv.at[j]], rows)
            pltpu.sync_copy(rows, acc.at[dstv.at[j]], add=True)
            return carry

        lax.fori_loop(0, _WCH, body, 0)

        plsc.subcore_barrier()
        pltpu.sync_copy(acc.at[pl.ds(s * _RPS, _RPS)],
                        p_hbm.at[c, pl.ds(s * _RPS, _RPS)])

    return agg


_agg80 = _make_agg(80)
_agg64 = _make_agg(64)


def _body_a(x_ref, wl_ref, e_ref, wr_ref, b_ref, y_ref, z_ref):
    xv = x_ref[...]
    y_ref[...] = jnp.dot(xv, wl_ref[...],
                         preferred_element_type=jnp.float32) + e_ref[...]
    z_ref[...] = jnp.dot(xv, wr_ref[...],
                         preferred_element_type=jnp.float32) + b_ref[...]


def _body_c(p_ref, z_ref, wl_ref, wr_ref, b_ref, y2_ref, z2_ref, inv_ref):
    ps = p_ref[0, : N] + p_ref[1, : N]
    col = lax.broadcasted_iota(jnp.int32, ps.shape, 1)
    deg = jnp.sum(jnp.where(col == H, ps, 0.0), axis=1, keepdims=True)
    inv = 1.0 / jnp.maximum(deg, 1.0)
    h = jnp.maximum(ps * inv + z_ref[...], 0.0)
    y2_ref[...] = jnp.dot(h, wl_ref[...], preferred_element_type=jnp.float32)
    z2_ref[...] = jnp.dot(h, wr_ref[...],
                          preferred_element_type=jnp.float32) + b_ref[...]
    inv_ref[...] = inv


def _body_d(p_ref, z_ref, inv_ref, wc_ref, bc_ref, o_ref):
    ps = p_ref[0, : N] + p_ref[1, : N]
    h = jnp.maximum(ps * inv_ref[...] + z_ref[...], 0.0)
    o_ref[...] = jnp.dot(h, wc_ref[...],
                         preferred_element_type=jnp.float32) + bc_ref[...]


def kernel(x, edge_index, Wl1, bl1, Wr1, Wl2, bl2, Wr2, Wc, bc):
    f32 = jnp.float32
    npad = _EPAD - E
    src = jnp.concatenate(
        [edge_index[0], jnp.zeros((npad,), jnp.int32)]).reshape(_NCHP, _CPB)
    dst = jnp.concatenate(
        [edge_index[1], jnp.full((npad,), N, jnp.int32)]).reshape(_NCHP, _CPB)

    Wp = H + 16
    Wl1p = jnp.pad(Wl1, ((0, 0), (0, Wp - H)))
    Wr1p = jnp.pad(Wr1, ((0, 0), (0, Wp - H)))
    e64 = jnp.zeros((1, Wp), f32).at[0, H].set(1.0)
    b1p = jnp.pad(bl1, (0, Wp - H)).reshape(1, Wp)
    Wl2p = jnp.pad(Wl2, ((0, Wp - H), (0, 0)))
    Wr2p = jnp.pad(Wr2, ((0, Wp - H), (0, 0)))
    b2 = bl2.reshape(1, H)
    Cp = 8
    Wcp = jnp.pad(Wc, ((0, 0), (0, Cp - Wc.shape[1])))
    bcp = jnp.pad(bc, (0, Cp - bc.shape[0])).reshape(1, Cp)

    y1p, z1p = pl.pallas_call(
        _body_a,
        out_shape=(jax.ShapeDtypeStruct((N, Wp), f32),
                   jax.ShapeDtypeStruct((N, Wp), f32)),
    )(x, Wl1p, e64, Wr1p, b1p)

    p1 = _agg80(y1p, src, dst)

    y2, z2, inv = pl.pallas_call(
        _body_c,
        out_shape=(jax.ShapeDtypeStruct((N, H), f32),
                   jax.ShapeDtypeStruct((N, H), f32),
                   jax.ShapeDtypeStruct((N, 1), f32)),
    )(p1, z1p, Wl2p, Wr2p, b2)

    p2 = _agg64(y2, src, dst)

    out = pl.pallas_call(
        _body_d,
        out_shape=jax.ShapeDtypeStruct((N, Cp), f32),
    )(p2, z2, inv, Wcp, bcp)

    return out[:, : Wc.shape[1]]


# speedup vs baseline: 5.2747x; 1.1177x over previous
import functools

import jax
import jax.numpy as jnp
from jax import lax
from jax.experimental import pallas as pl
from jax.experimental.pallas import tpu as pltpu
from jax.experimental.pallas import tpu_sc as plsc

N = 10000
E = 320000
D = 128
H = 64

_NC = 2
_NS = 16
_CPB = 128
_NCHP = 2560
_EPAD = _NCHP * _CPB
_WCH = _NCHP // (_NC * _NS)
_NR = 10112
_RPS = _NR // _NS


def _make_agg(W, _K):
    _G = _WCH // _K
    mesh = plsc.VectorSubcoreMesh(core_axis_name="c", subcore_axis_name="s")

    @functools.partial(
        pl.kernel,
        mesh=mesh,
        compiler_params=pltpu.CompilerParams(use_tc_tiling_on_sc=False),
        out_type=jax.ShapeDtypeStruct((_NC, _NR, W), jnp.float32),
        scratch_types=[
            pltpu.VMEM((_WCH, _CPB), jnp.int32),
            pltpu.VMEM((_WCH, _CPB), jnp.int32),
            pltpu.VMEM((2 * _K * _CPB, W), jnp.float32),
            pltpu.VMEM_SHARED((_NR, W), jnp.float32),
            pltpu.SemaphoreType.DMA,
            pltpu.SemaphoreType.DMA,
        ],
    )
    def agg(y_hbm, src_hbm, dst_hbm, p_hbm, srcv, dstv, rows, acc,
            sem_g, sem_s):
        c = lax.axis_index("c")
        s = lax.axis_index("s")
        wid = s * _NC + c

        zeros16 = jnp.zeros((16,), jnp.float32)

        def zbody(r, carry):
            for k2 in range(W // 16):
                rows[r, pl.ds(k2 * 16, 16)] = zeros16
            return carry

        lax.fori_loop(0, _CPB, zbody, 0)
        full = _RPS // _CPB
        rem = _RPS - full * _CPB
        for t in range(full):
            pltpu.sync_copy(rows.at[pl.ds(0, _CPB)],
                            acc.at[pl.ds(s * _RPS + t * _CPB, _CPB)])
        if rem:
            pltpu.sync_copy(rows.at[pl.ds(0, rem)],
                            acc.at[pl.ds(s * _RPS + full * _CPB, rem)])

        pltpu.sync_copy(src_hbm.at[pl.ds(wid * _WCH, _WCH)], srcv)
        pltpu.sync_copy(dst_hbm.at[pl.ds(wid * _WCH, _WCH)], dstv)

        plsc.subcore_barrier()

        def buf(g, b):
            return rows.at[pl.ds(((g % 2) * _K + b) * _CPB, _CPB)]

        def fire_gathers(g):
            for b in range(_K):
                pltpu.async_copy(y_hbm.at[srcv.at[g * _K + b]], buf(g, b),
                                 sem_g)

        def drain_gathers(g):
            for b in range(_K):
                pltpu.make_async_copy(y_hbm.at[srcv.at[0]], buf(g, b),
                                      sem_g).wait()

        def fire_scatters(g):
            for b in range(_K):
                pltpu.async_copy(buf(g, b), acc.at[dstv.at[g * _K + b]],
                                 sem_s, add=True)

        def drain_scatters(g):
            for b in range(_K):
                pltpu.make_async_copy(buf(g, b), acc.at[dstv.at[0]],
                                      sem_s).wait()

        fire_gathers(0)

        def body(g, carry):
            drain_gathers(g)

            @pl.when(g >= 1)
            def _():
                drain_scatters(g - 1)

            @pl.when(g + 1 < _G)
            def _():
                fire_gathers(g + 1)

            fire_scatters(g)
            return carry

        lax.fori_loop(0, _G, body, 0)
        drain_scatters(_G - 1)

        plsc.subcore_barrier()
        pltpu.sync_copy(acc.at[pl.ds(s * _RPS, _RPS)],
                        p_hbm.at[c, pl.ds(s * _RPS, _RPS)])

    return agg


_agg80 = _make_agg(80, 2)
_agg64 = _make_agg(64, 4)


def _body_a(x_ref, wl_ref, e_ref, wr_ref, b_ref, y_ref, z_ref):
    xv = x_ref[...]
    y_ref[...] = jnp.dot(xv, wl_ref[...],
                         preferred_element_type=jnp.float32) + e_ref[...]
    z_ref[...] = jnp.dot(xv, wr_ref[...],
                         preferred_element_type=jnp.float32) + b_ref[...]


def _body_c(p_ref, z_ref, wl_ref, wr_ref, b_ref, y2_ref, z2_ref, inv_ref):
    ps = p_ref[0, : N] + p_ref[1, : N]
    col = lax.broadcasted_iota(jnp.int32, ps.shape, 1)
    deg = jnp.sum(jnp.where(col == H, ps, 0.0), axis=1, keepdims=True)
    inv = 1.0 / jnp.maximum(deg, 1.0)
    h = jnp.maximum(ps * inv + z_ref[...], 0.0)
    y2_ref[...] = jnp.dot(h, wl_ref[...], preferred_element_type=jnp.float32)
    z2_ref[...] = jnp.dot(h, wr_ref[...],
                          preferred_element_type=jnp.float32) + b_ref[...]
    inv_ref[...] = inv


def _body_d(p_ref, z_ref, inv_ref, wc_ref, bc_ref, o_ref):
    ps = p_ref[0, : N] + p_ref[1, : N]
    h = jnp.maximum(ps * inv_ref[...] + z_ref[...], 0.0)
    o_ref[...] = jnp.dot(h, wc_ref[...],
                         preferred_element_type=jnp.float32) + bc_ref[...]


def kernel(x, edge_index, Wl1, bl1, Wr1, Wl2, bl2, Wr2, Wc, bc):
    f32 = jnp.float32
    npad = _EPAD - E
    src = jnp.concatenate(
        [edge_index[0], jnp.zeros((npad,), jnp.int32)]).reshape(_NCHP, _CPB)
    dst = jnp.concatenate(
        [edge_index[1], jnp.full((npad,), N, jnp.int32)]).reshape(_NCHP, _CPB)

    Wp = H + 16
    Wl1p = jnp.pad(Wl1, ((0, 0), (0, Wp - H)))
    Wr1p = jnp.pad(Wr1, ((0, 0), (0, Wp - H)))
    e64 = jnp.zeros((1, Wp), f32).at[0, H].set(1.0)
    b1p = jnp.pad(bl1, (0, Wp - H)).reshape(1, Wp)
    Wl2p = jnp.pad(Wl2, ((0, Wp - H), (0, 0)))
    Wr2p = jnp.pad(Wr2, ((0, Wp - H), (0, 0)))
    b2 = bl2.reshape(1, H)
    Cp = 8
    Wcp = jnp.pad(Wc, ((0, 0), (0, Cp - Wc.shape[1])))
    bcp = jnp.pad(bc, (0, Cp - bc.shape[0])).reshape(1, Cp)

    y1p, z1p = pl.pallas_call(
        _body_a,
        out_shape=(jax.ShapeDtypeStruct((N, Wp), f32),
                   jax.ShapeDtypeStruct((N, Wp), f32)),
    )(x, Wl1p, e64, Wr1p, b1p)

    p1 = _agg80(y1p, src, dst)

    y2, z2, inv = pl.pallas_call(
        _body_c,
        out_shape=(jax.ShapeDtypeStruct((N, H), f32),
                   jax.ShapeDtypeStruct((N, H), f32),
                   jax.ShapeDtypeStruct((N, 1), f32)),
    )(p1, z1p, Wl2p, Wr2p, b2)

    p2 = _agg64(y2, src, dst)

    out = pl.pallas_call(
        _body_d,
        out_shape=jax.ShapeDtypeStruct((N, Cp), f32),
    )(p2, z2, inv, Wcp, bcp)

    return out[:, : Wc.shape[1]]


# speedup vs baseline: 12.7153x; 2.4107x over previous
import functools

import jax
import jax.numpy as jnp
from jax import lax
from jax.experimental import pallas as pl
from jax.experimental.pallas import tpu as pltpu
from jax.experimental.pallas import tpu_sc as plsc

N = 10000
E = 320000
D = 128
H = 64

_NC = 2
_NS = 16
_CPB = 128
_NCHP = 2560
_EPAD = _NCHP * _CPB
_WCH = _NCHP // (_NC * _NS)
_HCH = _WCH // 2
_NR = 10016
_RPS = _NR // _NS
_TRS = N // _NS
_DW = 16


def _make_agg(with_deg):
    W = H
    mesh = plsc.VectorSubcoreMesh(core_axis_name="c", subcore_axis_name="s")
    out_type = [jax.ShapeDtypeStruct((_NC, _NR, W), jnp.float32)]
    scratch = [
        pltpu.VMEM((_HCH, _CPB), jnp.int32),
        pltpu.VMEM((_HCH, _CPB), jnp.int32),
        pltpu.VMEM((2 * _CPB, W), jnp.float32),
        pltpu.VMEM_SHARED((N, W), jnp.float32),
        pltpu.VMEM_SHARED((_NR, W), jnp.float32),
        pltpu.SemaphoreType.DMA,
        pltpu.SemaphoreType.DMA,
    ]
    if with_deg:
        out_type.append(jax.ShapeDtypeStruct((_NC, _NR, _DW), jnp.float32))
        scratch.append(pltpu.VMEM((_CPB, _DW), jnp.float32))
        scratch.append(pltpu.VMEM_SHARED((_NR, _DW), jnp.float32))

    @functools.partial(
        pl.kernel,
        mesh=mesh,
        compiler_params=pltpu.CompilerParams(use_tc_tiling_on_sc=False),
        out_type=tuple(out_type),
        scratch_types=scratch,
    )
    def agg(y_hbm, src_hbm, dst_hbm, p_hbm, *rest):
        if with_deg:
            pd_hbm, srcv, dstv, rows, ytab, acc, sem_g, sem_s, ones, dacc = rest
        else:
            srcv, dstv, rows, ytab, acc, sem_g, sem_s = rest
        c = lax.axis_index("c")
        s = lax.axis_index("s")
        wid = s * _NC + c

        pltpu.sync_copy(y_hbm.at[pl.ds(s * _TRS, _TRS)],
                        ytab.at[pl.ds(s * _TRS, _TRS)])

        zeros16 = jnp.zeros((16,), jnp.float32)

        def zbody(r, carry):
            for k2 in range(W // 16):
                rows[r, pl.ds(k2 * 16, 16)] = zeros16
            return carry

        lax.fori_loop(0, _CPB, zbody, 0)
        full = _RPS // _CPB
        rem = _RPS - full * _CPB
        for t in range(full):
            pltpu.sync_copy(rows.at[pl.ds(0, _CPB)],
                            acc.at[pl.ds(s * _RPS + t * _CPB, _CPB)])
        if rem:
            pltpu.sync_copy(rows.at[pl.ds(0, rem)],
                            acc.at[pl.ds(s * _RPS + full * _CPB, rem)])

        if with_deg:
            def dbody(r, carry):
                ones[r, pl.ds(0, 16)] = zeros16
                return carry

            lax.fori_loop(0, _CPB, dbody, 0)
            for t in range(full):
                pltpu.sync_copy(ones.at[pl.ds(0, _CPB)],
                                dacc.at[pl.ds(s * _RPS + t * _CPB, _CPB)])
            if rem:
                pltpu.sync_copy(ones.at[pl.ds(0, rem)],
                                dacc.at[pl.ds(s * _RPS + full * _CPB, rem)])

            ones16 = jnp.full((16,), 1.0, jnp.float32)

            def obody(r, carry):
                ones[r, pl.ds(0, 16)] = ones16
                return carry

            lax.fori_loop(0, _CPB, obody, 0)

        plsc.subcore_barrier()

        def buf(g):
            return rows.at[pl.ds((g % 2) * _CPB, _CPB)]

        def fire_gather(g, jj):
            pltpu.async_copy(ytab.at[srcv.at[jj]], buf(g), sem_g)

        def drain_gather(g):
            pltpu.make_async_copy(ytab.at[srcv.at[0]], buf(g), sem_g).wait()

        def fire_scatter(g, jj):
            pltpu.async_copy(buf(g), acc.at[dstv.at[jj]], sem_s, add=True)
            if with_deg:
                pltpu.async_copy(ones, dacc.at[dstv.at[jj]], sem_s, add=True)

        def drain_scatter(g):
            pltpu.make_async_copy(buf(g), acc.at[dstv.at[0]], sem_s).wait()
            if with_deg:
                pltpu.make_async_copy(ones, dacc.at[dstv.at[0]],
                                      sem_s).wait()

        for half in range(2):
            pltpu.sync_copy(
                src_hbm.at[pl.ds(wid * _WCH + half * _HCH, _HCH)], srcv)
            pltpu.sync_copy(
                dst_hbm.at[pl.ds(wid * _WCH + half * _HCH, _HCH)], dstv)

            fire_gather(0, 0)

            def body(g, carry):
                drain_gather(g)

                @pl.when(g >= 1)
                def _():
                    drain_scatter(g - 1)

                @pl.when(g + 1 < _HCH)
                def _():
                    fire_gather(g + 1, g + 1)

                fire_scatter(g, g)
                return carry

            lax.fori_loop(0, _HCH, body, 0)
            drain_scatter(_HCH - 1)

        plsc.subcore_barrier()
        pltpu.sync_copy(acc.at[pl.ds(s * _RPS, _RPS)],
                        p_hbm.at[c, pl.ds(s * _RPS, _RPS)])
        if with_deg:
            pltpu.sync_copy(dacc.at[pl.ds(s * _RPS, _RPS)],
                            pd_hbm.at[c, pl.ds(s * _RPS, _RPS)])

    return agg


_agg_l1 = _make_agg(True)
_agg_l2 = _make_agg(False)


def _body_a(x_ref, wl_ref, wr_ref, b_ref, y_ref, z_ref):
    xv = x_ref[...]
    y_ref[...] = jnp.dot(xv, wl_ref[...], preferred_element_type=jnp.float32)
    z_ref[...] = jnp.dot(xv, wr_ref[...],
                         preferred_element_type=jnp.float32) + b_ref[...]


def _body_c(p_ref, pd_ref, z_ref, wl_ref, wr_ref, b_ref,
            y2_ref, z2_ref, inv_ref):
    ps = p_ref[0, : N] + p_ref[1, : N]
    degs = pd_ref[0, : N] + pd_ref[1, : N]
    deg = jnp.sum(degs, axis=1, keepdims=True) * (1.0 / _DW)
    inv = 1.0 / jnp.maximum(deg, 1.0)
    h = jnp.maximum(ps * inv + z_ref[...], 0.0)
    y2_ref[...] = jnp.dot(h, wl_ref[...], preferred_element_type=jnp.float32)
    z2_ref[...] = jnp.dot(h, wr_ref[...],
                          preferred_element_type=jnp.float32) + b_ref[...]
    inv_ref[...] = inv


def _body_d(p_ref, z_ref, inv_ref, wc_ref, bc_ref, o_ref):
    ps = p_ref[0, : N] + p_ref[1, : N]
    h = jnp.maximum(ps * inv_ref[...] + z_ref[...], 0.0)
    o_ref[...] = jnp.dot(h, wc_ref[...],
                         preferred_element_type=jnp.float32) + bc_ref[...]


def kernel(x, edge_index, Wl1, bl1, Wr1, Wl2, bl2, Wr2, Wc, bc):
    f32 = jnp.float32
    npad = _EPAD - E
    src = jnp.concatenate(
        [edge_index[0], jnp.zeros((npad,), jnp.int32)]).reshape(_NCHP, _CPB)
    dst = jnp.concatenate(
        [edge_index[1], jnp.full((npad,), N, jnp.int32)]).reshape(_NCHP, _CPB)

    b1 = bl1.reshape(1, H)
    b2 = bl2.reshape(1, H)
    Cp = 8
    Wcp = jnp.pad(Wc, ((0, 0), (0, Cp - Wc.shape[1])))
    bcp = jnp.pad(bc, (0, Cp - bc.shape[0])).reshape(1, Cp)

    y1, z1 = pl.pallas_call(
        _body_a,
        out_shape=(jax.ShapeDtypeStruct((N, H), f32),
                   jax.ShapeDtypeStruct((N, H), f32)),
    )(x, Wl1, Wr1, b1)

    p1, pdeg = _agg_l1(y1, src, dst)

    y2, z2, inv = pl.pallas_call(
        _body_c,
        out_shape=(jax.ShapeDtypeStruct((N, H), f32),
                   jax.ShapeDtypeStruct((N, H), f32),
                   jax.ShapeDtypeStruct((N, 1), f32)),
    )(p1, pdeg, z1, Wl2, Wr2, b2)

    p2, = _agg_l2(y2, src, dst)

    out = pl.pallas_call(
        _body_d,
        out_shape=jax.ShapeDtypeStruct((N, Cp), f32),
    )(p2, z2, inv, Wcp, bcp)

    return out[:, : Wc.shape[1]]


# speedup vs baseline: 13.6667x; 1.0748x over previous
import functools

import jax
import jax.numpy as jnp
from jax import lax
from jax.experimental import pallas as pl
from jax.experimental.pallas import tpu as pltpu
from jax.experimental.pallas import tpu_sc as plsc

N = 10000
E = 320000
D = 128
H = 64

_NC = 2
_NS = 16
_NW = _NC * _NS
_CPB = 128
_NCH = E // _CPB
_WCH = 80
_NR = 10016
_RPS = _NR // _NS
_TRS = N // _NS
_DW = 16


def _make_agg(with_deg):
    W = H
    mesh = plsc.VectorSubcoreMesh(core_axis_name="c", subcore_axis_name="s")
    out_type = [jax.ShapeDtypeStruct((_NC, _NR, W), jnp.float32)]
    scratch = [
        pltpu.VMEM((_WCH, _CPB), jnp.int32),
        pltpu.VMEM((_WCH, _CPB), jnp.int32),
        pltpu.VMEM((2 * _CPB, W), jnp.float32),
        pltpu.VMEM_SHARED((N, W), jnp.float32),
        pltpu.VMEM_SHARED((_NR, W), jnp.float32),
        pltpu.SemaphoreType.DMA,
        pltpu.SemaphoreType.DMA,
    ]
    if with_deg:
        out_type.append(jax.ShapeDtypeStruct((_NC, _NR, _DW), jnp.float32))
        scratch.append(pltpu.VMEM((_CPB, _DW), jnp.float32))
        scratch.append(pltpu.VMEM_SHARED((_NR, _DW), jnp.float32))

    @functools.partial(
        pl.kernel,
        mesh=mesh,
        compiler_params=pltpu.CompilerParams(use_tc_tiling_on_sc=False),
        out_type=tuple(out_type),
        scratch_types=scratch,
    )
    def agg(y_hbm, ei_hbm, p_hbm, *rest):
        if with_deg:
            pd_hbm, srcv, dstv, rows, ytab, acc, sem_g, sem_s, ones, dacc = rest
        else:
            srcv, dstv, rows, ytab, acc, sem_g, sem_s = rest
        c = lax.axis_index("c")
        s = lax.axis_index("s")
        wid = s * _NC + c

        pltpu.sync_copy(y_hbm.at[pl.ds(s * _TRS, _TRS)],
                        ytab.at[pl.ds(s * _TRS, _TRS)])

        base = wid * _WCH
        nch = jnp.minimum(_WCH, _NCH - base)
        base_l = jnp.minimum(base, _NCH - _WCH)
        off = base - base_l
        pltpu.sync_copy(ei_hbm.at[0, pl.ds(base_l, _WCH)], srcv)
        pltpu.sync_copy(ei_hbm.at[1, pl.ds(base_l, _WCH)], dstv)

        zeros16 = jnp.zeros((16,), jnp.float32)

        def zbody(r, carry):
            for k2 in range(W // 16):
                rows[r, pl.ds(k2 * 16, 16)] = zeros16
            return carry

        lax.fori_loop(0, _CPB, zbody, 0)
        full = _RPS // _CPB
        rem = _RPS - full * _CPB
        for t in range(full):
            pltpu.sync_copy(rows.at[pl.ds(0, _CPB)],
                            acc.at[pl.ds(s * _RPS + t * _CPB, _CPB)])
        if rem:
            pltpu.sync_copy(rows.at[pl.ds(0, rem)],
                            acc.at[pl.ds(s * _RPS + full * _CPB, rem)])

        if with_deg:
            def dbody(r, carry):
                ones[r, pl.ds(0, 16)] = zeros16
                return carry

            lax.fori_loop(0, _CPB, dbody, 0)
            for t in range(full):
                pltpu.sync_copy(ones.at[pl.ds(0, _CPB)],
                                dacc.at[pl.ds(s * _RPS + t * _CPB, _CPB)])
            if rem:
                pltpu.sync_copy(ones.at[pl.ds(0, rem)],
                                dacc.at[pl.ds(s * _RPS + full * _CPB, rem)])

            ones16 = jnp.full((16,), 1.0, jnp.float32)

            def obody(r, carry):
                ones[r, pl.ds(0, 16)] = ones16
                return carry

            lax.fori_loop(0, _CPB, obody, 0)

        plsc.subcore_barrier()

        def buf(g):
            return rows.at[pl.ds((g % 2) * _CPB, _CPB)]

        def fire_gather(g):
            pltpu.async_copy(ytab.at[srcv.at[g + off]], buf(g), sem_g)

        def drain_gather(g):
            pltpu.make_async_copy(ytab.at[srcv.at[0]], buf(g), sem_g).wait()

        def fire_scatter(g):
            pltpu.async_copy(buf(g), acc.at[dstv.at[g + off]], sem_s,
                             add=True)
            if with_deg:
                pltpu.async_copy(ones, dacc.at[dstv.at[g + off]], sem_s,
                                 add=True)

        def drain_scatter(g):
            pltpu.make_async_copy(buf(g), acc.at[dstv.at[0]], sem_s).wait()
            if with_deg:
                pltpu.make_async_copy(ones, dacc.at[dstv.at[0]],
                                      sem_s).wait()

        fire_gather(0)

        def body(g, carry):
            drain_gather(g)

            @pl.when(g >= 1)
            def _():
                drain_scatter(g - 1)

            @pl.when(g + 1 < nch)
            def _():
                fire_gather(g + 1)

            fire_scatter(g)
            return carry

        lax.fori_loop(0, nch, body, 0)
        drain_scatter(nch - 1)

        plsc.subcore_barrier()
        pltpu.sync_copy(acc.at[pl.ds(s * _RPS, _RPS)],
                        p_hbm.at[c, pl.ds(s * _RPS, _RPS)])
        if with_deg:
            pltpu.sync_copy(dacc.at[pl.ds(s * _RPS, _RPS)],
                            pd_hbm.at[c, pl.ds(s * _RPS, _RPS)])

    return agg


_agg_l1 = _make_agg(True)
_agg_l2 = _make_agg(False)


def _body_a(x_ref, wl_ref, y_ref):
    y_ref[...] = jnp.dot(x_ref[...], wl_ref[...],
                         preferred_element_type=jnp.float32)


def _body_c(p_ref, pd_ref, x_ref, wr1_ref, b1_ref, wl_ref, wr_ref, b_ref,
            y2_ref, z2_ref, inv_ref):
    ps = p_ref[0, : N] + p_ref[1, : N]
    degs = pd_ref[0, : N] + pd_ref[1, : N]
    deg = jnp.sum(degs, axis=1, keepdims=True) * (1.0 / _DW)
    inv = 1.0 / jnp.maximum(deg, 1.0)
    z1 = jnp.dot(x_ref[...], wr1_ref[...],
                 preferred_element_type=jnp.float32) + b1_ref[...]
    h = jnp.maximum(ps * inv + z1, 0.0)
    y2_ref[...] = jnp.dot(h, wl_ref[...], preferred_element_type=jnp.float32)
    z2_ref[...] = jnp.dot(h, wr_ref[...],
                          preferred_element_type=jnp.float32) + b_ref[...]
    inv_ref[...] = inv


def _body_d(p_ref, z_ref, inv_ref, wc_ref, bc_ref, o_ref):
    ps = p_ref[0, : N] + p_ref[1, : N]
    h = jnp.maximum(ps * inv_ref[...] + z_ref[...], 0.0)
    o_ref[...] = jnp.dot(h, wc_ref[...],
                         preferred_element_type=jnp.float32) + bc_ref[...]


def kernel(x, edge_index, Wl1, bl1, Wr1, Wl2, bl2, Wr2, Wc, bc):
    f32 = jnp.float32
    ei = edge_index.reshape(2, _NCH, _CPB)

    b1 = bl1.reshape(1, H)
    b2 = bl2.reshape(1, H)
    Cp = 8
    Wcp = jnp.pad(Wc, ((0, 0), (0, Cp - Wc.shape[1])))
    bcp = jnp.pad(bc, (0, Cp - bc.shape[0])).reshape(1, Cp)

    y1 = pl.pallas_call(
        _body_a,
        out_shape=jax.ShapeDtypeStruct((N, H), f32),
    )(x, Wl1)

    p1, pdeg = _agg_l1(y1, ei)

    y2, z2, inv = pl.pallas_call(
        _body_c,
        out_shape=(jax.ShapeDtypeStruct((N, H), f32),
                   jax.ShapeDtypeStruct((N, H), f32),
                   jax.ShapeDtypeStruct((N, 1), f32)),
    )(p1, pdeg, x, Wr1, b1, Wl2, Wr2, b2)

    p2, = _agg_l2(y2, ei)

    out = pl.pallas_call(
        _body_d,
        out_shape=jax.ShapeDtypeStruct((N, Cp), f32),
    )(p2, z2, inv, Wcp, bcp)

    return out[:, : Wc.shape[1]]
